# Initial kernel scaffold; baseline (speedup 1.0000x reference)
import jax
import jax.numpy as jnp
from jax.experimental import pallas as pl


def kernel(x, target):
    raise NotImplementedError("write your pallas kernel here")


# speedup vs baseline: 6.4692x; 6.4692x over previous
import functools
import math

import jax
import jax.numpy as jnp
from jax import lax
from jax.experimental import pallas as pl
from jax.experimental.pallas import tpu as pltpu

VOCAB = 32768
SMOOTH = 0.1
CONF = 1.0 - SMOOTH
EPS = SMOOTH / (VOCAB - 2)
K_CONST = (VOCAB - 2) * EPS * math.log(EPS) + CONF * math.log(CONF)

BLK_R = 256
BLK_V = 4096


def _tc_body(t_ref, x_ref, out_ref, acc_ref):
    i = pl.program_id(0)
    j = pl.program_id(1)
    ni = pl.num_programs(0)
    nj = pl.num_programs(1)

    @pl.when((i == 0) & (j == 0))
    def _init():
        acc_ref[0] = 0.0
        acc_ref[1] = 0.0
        acc_ref[2] = 0.0
        acc_ref[3] = 0.0

    t = t_ref[...]
    w = (t != 0).astype(jnp.float32)
    xs = x_ref[...]
    rs = jnp.sum(xs, axis=1, keepdims=True)
    acc_ref[0] += jnp.sum(rs * w)

    col = lax.broadcasted_iota(jnp.int32, (BLK_R, BLK_V), 1) + j * BLK_V
    st = jnp.where((col == t) & (t != 0), xs, 0.0)
    acc_ref[3] += jnp.sum(st)

    @pl.when(j == 0)
    def _col0():
        acc_ref[1] += jnp.sum(xs[:, 0:1] * w)
        acc_ref[2] += jnp.sum(w)

    @pl.when((i == ni - 1) & (j == nj - 1))
    def _fin():
        total = acc_ref[0]
        s0 = acc_ref[1]
        cnt = acc_ref[2]
        st_sum = acc_ref[3]
        out_ref[0] = (cnt * K_CONST - EPS * (total - s0)
                      - (CONF - EPS) * st_sum)


@jax.jit
def _loss_tc(x, t2d):
    n = x.shape[0]
    grid = (n // BLK_R, VOCAB // BLK_V)
    res = pl.pallas_call(
        _tc_body,
        grid=grid,
        in_specs=[
            pl.BlockSpec((BLK_R, 1), lambda i, j: (i, 0)),
            pl.BlockSpec((BLK_R, BLK_V), lambda i, j: (i, j)),
        ],
        out_specs=pl.BlockSpec(memory_space=pltpu.SMEM),
        out_shape=jax.ShapeDtypeStruct((1,), jnp.float32),
        scratch_shapes=[pltpu.SMEM((4,), jnp.float32)],
    )(t2d, x)
    return res[0]


def kernel(x, target):
    t2d = target.astype(jnp.int32).reshape(-1, 1)
    return _loss_tc(x, t2d)
